# baseline (speedup 1.0000x reference)
import functools
import jax
import jax.numpy as jnp
from jax import lax
from jax.experimental import pallas as pl
from jax.experimental.pallas import tpu as pltpu
from jax.experimental.pallas import tpu_sc as plsc

B = 8192
D = 1024
H = 256
E = 3
BM = 512

NC = 2
NS = 16
L = 16
NW = NC * NS
RPW = B // NW
C = 8
NCH = RPW // C
WB = 3 * L


def _gate_kernel(cs_ref, na_ref, w1ag_ref, w1b_ref, u_ref, c1_ref, w2_ref,
                 b2_ref, wts_ref, wb_ref):
    cs = cs_ref[...]
    mu = jnp.mean(cs, axis=1, keepdims=True)
    xc = cs - mu
    var = jnp.mean(xc * xc, axis=1, keepdims=True)
    s = jax.lax.rsqrt(var + 1e-5)
    t = (jnp.dot(cs.astype(jnp.bfloat16), w1ag_ref[...],
                 preferred_element_type=jnp.float32) * s
         + jnp.dot(na_ref[...].astype(jnp.bfloat16), w1b_ref[...],
                   preferred_element_type=jnp.float32))
    h = t - (s * mu) * u_ref[...] + c1_ref[...]
    h = 0.5 * h * (1.0 + jax.lax.erf(h * 0.7071067811865476))
    logits = jnp.dot(h, w2_ref[...], preferred_element_type=jnp.float32) + b2_ref[...]
    m = jnp.max(logits, axis=1, keepdims=True)
    ex = jnp.exp(logits - m)
    w = ex / jnp.sum(ex, axis=1, keepdims=True)
    wts_ref[...] = w
    wb_ref[...] = jnp.concatenate(
        [jnp.broadcast_to(w[:, 0:1], (BM, L)),
         jnp.broadcast_to(w[:, 1:2], (BM, L)),
         jnp.broadcast_to(w[:, 2:3], (BM, L))], axis=1)


def _gate(cs, na, w1ag, w1b, u, c1, W2, b2r):
    grid = (B // BM,)
    row = lambda i: (i, 0)
    rep = lambda i: (0, 0)
    return pl.pallas_call(
        _gate_kernel,
        grid=grid,
        in_specs=[
            pl.BlockSpec((BM, D), row),
            pl.BlockSpec((BM, D), row),
            pl.BlockSpec((D, H), rep),
            pl.BlockSpec((D, H), rep),
            pl.BlockSpec((1, H), rep),
            pl.BlockSpec((1, H), rep),
            pl.BlockSpec((H, E), rep),
            pl.BlockSpec((1, E), rep),
        ],
        out_specs=[
            pl.BlockSpec((BM, E), row),
            pl.BlockSpec((BM, WB), row),
        ],
        out_shape=[
            jax.ShapeDtypeStruct((B, E), jnp.float32),
            jax.ShapeDtypeStruct((B, WB), jnp.float32),
        ],
    )(cs, na, w1ag, w1b, u, c1, W2, b2r)


_sc_mesh = plsc.VectorSubcoreMesh(core_axis_name="c", subcore_axis_name="s")


@functools.partial(
    pl.kernel,
    out_type=jax.ShapeDtypeStruct((B, D), jnp.float32),
    mesh=_sc_mesh,
    scratch_types=[
        pltpu.VMEM((2, C, D), jnp.float32),
        pltpu.VMEM((2, C, D), jnp.float32),
        pltpu.VMEM((2, C, D), jnp.float32),
        pltpu.VMEM((2, C, WB), jnp.float32),
        pltpu.VMEM((2, C, D), jnp.float32),
        pltpu.SemaphoreType.DMA,
        pltpu.SemaphoreType.DMA,
        pltpu.SemaphoreType.DMA,
        pltpu.SemaphoreType.DMA,
    ],
)
def _sc_combine(e0_hbm, e1_hbm, e2_hbm, wb_hbm, out_hbm,
                e0v, e1v, e2v, wbv, outv, isem0, isem1, osem0, osem1):
    wid = lax.axis_index("s") * NC + lax.axis_index("c")
    base = wid * RPW
    isems = (isem0, isem1)
    osems = (osem0, osem1)

    def in_copies(g, b):
        r0 = base + g * C
        sem = isems[b]
        return (pltpu.make_async_copy(e0_hbm.at[pl.ds(r0, C)], e0v.at[b], sem),
                pltpu.make_async_copy(e1_hbm.at[pl.ds(r0, C)], e1v.at[b], sem),
                pltpu.make_async_copy(e2_hbm.at[pl.ds(r0, C)], e2v.at[b], sem),
                pltpu.make_async_copy(wb_hbm.at[pl.ds(r0, C)], wbv.at[b], sem))

    def out_copy(g, b):
        r0 = base + g * C
        return pltpu.make_async_copy(outv.at[b], out_hbm.at[pl.ds(r0, C)],
                                     osems[b])

    def start_in(g, b):
        for cp in in_copies(g, b):
            cp.start()

    def wait_in(g, b):
        for cp in in_copies(g, b):
            cp.wait()

    def compute_chunk(b):
        def row_body(r, _):
            w0 = wbv[b, r, 0:L]
            w1 = wbv[b, r, L:2 * L]
            w2 = wbv[b, r, 2 * L:3 * L]

            def vec_body(j, _):
                sl = pl.ds(j * L, L)
                outv[b, r, sl] = (w0 * e0v[b, r, sl] + w1 * e1v[b, r, sl]
                                  + w2 * e2v[b, r, sl])
                return 0

            lax.fori_loop(0, D // L, vec_body, 0, unroll=4)
            return 0

        lax.fori_loop(0, C, row_body, 0)

    start_in(0, 0)
    start_in(1, 1)

    def super_body(tt, _):
        for b in range(2):
            g = 2 * tt + b
            wait_in(g, b)

            @pl.when(tt > 0)
            def _():
                out_copy(g - 2, b).wait()

            compute_chunk(b)
            out_copy(g, b).start()

            @pl.when(tt < NCH // 2 - 1)
            def _():
                start_in(g + 2, b)
        return 0

    lax.fori_loop(0, NCH // 2, super_body, 0)
    out_copy(NCH - 2, 0).wait()
    out_copy(NCH - 1, 1).wait()


def kernel(current_state, neighbor_activity, expert_out_0, expert_out_1, expert_out_2, ln_gamma, ln_beta, W1, b1, W2, b2):
    w1a = W1[:D]
    w1ag = (w1a * ln_gamma[:, None]).astype(jnp.bfloat16)
    w1b = W1[D:].astype(jnp.bfloat16)
    u = (ln_gamma @ w1a).reshape(1, H)
    c1 = (ln_beta @ w1a + b1).reshape(1, H)
    b2r = b2.reshape(1, E)

    wts, wb = _gate(current_state, neighbor_activity, w1ag, w1b, u, c1, W2, b2r)
    out = _sc_combine(expert_out_0, expert_out_1, expert_out_2, wb)
    return out, wts
